# baseline (device time: 117041 ns/iter reference)
import jax
import jax.numpy as jnp
from jax import lax
from jax.experimental import pallas as pl
from jax.experimental.pallas import tpu as pltpu

N_CHUNKS = 32


def kernel(x):
    _, m, n_half = x.shape
    n = 2 * n_half
    mc = m // N_CHUNKS

    def body(x_ref, out_ref, commx, sx, rx):
        my_x = lax.axis_index("x")
        my_y = lax.axis_index("y")
        x_nbr = (1 - my_x, my_y)
        y_nbr = (my_x, 1 - my_y)

        barrier_sem = pltpu.get_barrier_semaphore()
        for nbr in (x_nbr, y_nbr):
            pl.semaphore_signal(
                barrier_sem, inc=1,
                device_id=nbr, device_id_type=pl.DeviceIdType.MESH,
            )
        pl.semaphore_wait(barrier_sem, 2)

        rdmas_x = []
        for i in range(N_CHUNKS):
            r = pltpu.make_async_remote_copy(
                src_ref=x_ref.at[0, pl.ds(i * mc, mc), :],
                dst_ref=commx.at[pl.ds(i * mc, mc), :],
                send_sem=sx.at[i],
                recv_sem=rx.at[i],
                device_id=x_nbr,
                device_id_type=pl.DeviceIdType.MESH,
            )
            r.start()
            rdmas_x.append(r)

        for i in range(N_CHUNKS):
            rdmas_x[i].wait()
            rows = pl.ds(i * mc, mc)
            col = pl.ds(my_y * n_half, n_half)
            ocol = pl.ds((1 - my_y) * n_half, n_half)
            out_ref[rows, col] = x_ref[0, rows, :] + commx[rows, :]
            out_ref[rows, ocol] = commx[rows, :]

    return pl.pallas_call(
        body,
        out_shape=jax.ShapeDtypeStruct((m, n), jnp.float32),
        in_specs=[pl.BlockSpec(memory_space=pltpu.VMEM)],
        out_specs=pl.BlockSpec(memory_space=pltpu.VMEM),
        scratch_shapes=[
            pltpu.VMEM((m, n_half), jnp.float32),
            pltpu.SemaphoreType.DMA((N_CHUNKS,)),
            pltpu.SemaphoreType.DMA((N_CHUNKS,)),
        ],
        compiler_params=pltpu.CompilerParams(
            collective_id=0,
            vmem_limit_bytes=96 * 1024 * 1024,
        ),
    )(x)


# device time: 114121 ns/iter; 1.0256x vs baseline; 1.0256x over previous
import jax
import jax.numpy as jnp
from jax import lax
from jax.experimental import pallas as pl
from jax.experimental.pallas import tpu as pltpu

CHUNK_ROWS = [16, 32] + [64] * 30 + [32, 32, 16]
N_CHUNKS = len(CHUNK_ROWS)
CHUNK_OFF = [sum(CHUNK_ROWS[:i]) for i in range(N_CHUNKS)]


def kernel(x):
    _, m, n_half = x.shape
    n = 2 * n_half
    assert sum(CHUNK_ROWS) == m

    def body(x_ref, out_ref, xv, commx, red, sx, rx, sy, ry, cp, cpin):
        my_x = lax.axis_index("x")
        my_y = lax.axis_index("y")
        x_nbr = (1 - my_x, my_y)
        y_nbr = (my_x, 1 - my_y)

        barrier_sem = pltpu.get_barrier_semaphore()
        for nbr in (x_nbr, y_nbr):
            pl.semaphore_signal(
                barrier_sem, inc=1,
                device_id=nbr, device_id_type=pl.DeviceIdType.MESH,
            )
        in_copies = []
        for i in range(N_CHUNKS):
            c = pltpu.make_async_copy(
                x_ref.at[0, pl.ds(CHUNK_OFF[i], CHUNK_ROWS[i]), :],
                xv.at[pl.ds(CHUNK_OFF[i], CHUNK_ROWS[i]), :],
                cpin.at[i],
            )
            c.start()
            in_copies.append(c)

        pl.semaphore_wait(barrier_sem, 2)

        rdmas_x = []
        for i in range(N_CHUNKS):
            in_copies[i].wait()
            r = pltpu.make_async_remote_copy(
                src_ref=xv.at[pl.ds(CHUNK_OFF[i], CHUNK_ROWS[i]), :],
                dst_ref=commx.at[pl.ds(CHUNK_OFF[i], CHUNK_ROWS[i]), :],
                send_sem=sx.at[i],
                recv_sem=rx.at[i],
                device_id=x_nbr,
                device_id_type=pl.DeviceIdType.MESH,
            )
            r.start()
            rdmas_x.append(r)

        col = pl.ds(my_y * n_half, n_half)
        rdmas_y = []
        copies = []
        for i in range(N_CHUNKS):
            rdmas_x[i].wait()
            rows = pl.ds(CHUNK_OFF[i], CHUNK_ROWS[i])
            red[rows, :] = xv[rows, :] + commx[rows, :]
            c = pltpu.make_async_copy(
                red.at[rows, :], out_ref.at[rows, col], cp.at[i]
            )
            c.start()
            copies.append(c)
            r = pltpu.make_async_remote_copy(
                src_ref=red.at[rows, :],
                dst_ref=out_ref.at[rows, col],
                send_sem=sy.at[i],
                recv_sem=ry.at[i],
                device_id=y_nbr,
                device_id_type=pl.DeviceIdType.MESH,
            )
            r.start()
            rdmas_y.append(r)

        for i in range(N_CHUNKS):
            copies[i].wait()
            rdmas_y[i].wait()

    return pl.pallas_call(
        body,
        out_shape=jax.ShapeDtypeStruct((m, n), jnp.float32),
        in_specs=[pl.BlockSpec(memory_space=pl.ANY)],
        out_specs=pl.BlockSpec(memory_space=pl.ANY),
        scratch_shapes=[
            pltpu.VMEM((m, n_half), jnp.float32),
            pltpu.VMEM((m, n_half), jnp.float32),
            pltpu.VMEM((m, n_half), jnp.float32),
            pltpu.SemaphoreType.DMA((N_CHUNKS,)),
            pltpu.SemaphoreType.DMA((N_CHUNKS,)),
            pltpu.SemaphoreType.DMA((N_CHUNKS,)),
            pltpu.SemaphoreType.DMA((N_CHUNKS,)),
            pltpu.SemaphoreType.DMA((N_CHUNKS,)),
            pltpu.SemaphoreType.DMA((N_CHUNKS,)),
        ],
        compiler_params=pltpu.CompilerParams(
            collective_id=0,
            vmem_limit_bytes=96 * 1024 * 1024,
        ),
    )(x)
